# baseline (speedup 1.0000x reference)
import jax
import jax.numpy as jnp
from jax import lax
from jax.experimental import pallas as pl
from jax.experimental.pallas import tpu as pltpu
from jax.experimental.pallas import tpu_sc as plsc

NN = 10000
EE = 320000
NH = 4
CC = 32
HID1 = NH * CC
HH = HID1 // 2
HID2 = 64
NG = 64

NC = 2
NS = 16
LANES = 16
NW = NC * NS
EPT = EE // NW
CH = 80
NCHK = EPT // CH
SLAB = 624
DW = 16


def _a_body(x_ref, w_ref, ab_ref, xw0_ref, xw1_ref, a16_0_ref, a16_1_ref,
            se_ref, m0_ref, m1_ref):
    xw = jnp.dot(x_ref[...], w_ref[...], preferred_element_type=jnp.float32)
    xw0_ref[...] = xw[:, 0:HH]
    xw1_ref[...] = xw[:, HH:HID1]
    a8 = jnp.dot(xw, ab_ref[...], preferred_element_type=jnp.float32)
    z1 = jnp.zeros((NN, 1), jnp.float32)
    z11 = jnp.zeros((NN, 11), jnp.float32)
    a16_0_ref[...] = jnp.concatenate(
        [z1, a8[:, 0:2], a8[:, 4:6], z11], axis=1)
    a16_1_ref[...] = jnp.concatenate(
        [z1, a8[:, 2:4], a8[:, 6:8], z11], axis=1)
    amax = jnp.max(a8, axis=0, keepdims=True)
    m = amax[:, 0:4] + amax[:, 4:8]
    es = a8[:, 0:4] + a8[:, 4:8]
    es = jnp.where(es >= 0, es, es * 0.2)
    exs = jnp.exp(es - m)
    se_ref[...] = jnp.concatenate([exs, jnp.zeros_like(exs)], axis=1)
    ml = jnp.tile(m.reshape(4, 1), (1, LANES))
    m0_ref[...] = ml[0:2]
    m1_ref[...] = ml[2:4]


def _stage_a(x, gat_w, ab):
    return pl.pallas_call(
        _a_body,
        out_shape=(
            jax.ShapeDtypeStruct((NN, HH), jnp.float32),
            jax.ShapeDtypeStruct((NN, HH), jnp.float32),
            jax.ShapeDtypeStruct((NN, 16), jnp.float32),
            jax.ShapeDtypeStruct((NN, 16), jnp.float32),
            jax.ShapeDtypeStruct((NN, 8), jnp.float32),
            jax.ShapeDtypeStruct((2, LANES), jnp.float32),
            jax.ShapeDtypeStruct((2, LANES), jnp.float32),
        ),
    )(x, gat_w, ab)


def _gat_edge_body(src_hbm, dst_hbm, xw0_hbm, xw1_hbm, a0_hbm, a1_hbm,
                   m0_hbm, m1_hbm,
                   onum0_hbm, oden0_hbm, onum1_hbm, oden1_hbm,
                   src_v, dst_v, asrc_g, adst_g, m_v, rows_v, exb_v,
                   rows_w, asrc_h, adst_h, exb_w,
                   accn_s, accd_s, sem, sem2, ssem, ssem2):
    c = lax.axis_index("c")
    s = lax.axis_index("s")
    wid = c * NS + s
    base = s * SLAB

    pltpu.sync_copy(src_hbm.at[wid], src_v)
    pltpu.sync_copy(dst_hbm.at[wid], dst_v)
    pltpu.sync_copy(m0_hbm, m_v.at[pl.ds(0, 2)])
    pltpu.sync_copy(m1_hbm, m_v.at[pl.ds(2, 2)])

    zeros16 = jnp.zeros((LANES,), jnp.float32)
    iota16 = lax.iota(jnp.int32, LANES)
    ones16 = jnp.ones((LANES,), jnp.float32)

    def _zero_exb(r, _):
        exb_v[r, pl.ds(0, LANES)] = zeros16
        exb_w[r, pl.ds(0, LANES)] = zeros16
        return 0
    lax.fori_loop(0, CH, _zero_exb, 0)

    for g in range(CH // LANES):
        for xb in (exb_v, exb_w):
            plsc.store_scatter(
                xb, [g * LANES + iota16, jnp.full((LANES,), 3, jnp.int32)],
                ones16)

    bufs = ((rows_v, asrc_g, adst_g, exb_v, sem, ssem),
            (rows_w, asrc_h, adst_h, exb_w, sem2, ssem2))
    passes = ((xw0_hbm, a0_hbm, onum0_hbm, oden0_hbm),
              (xw1_hbm, a1_hbm, onum1_hbm, oden1_hbm))

    for p in range(2):
        xw_hbm, a16_hbm, onum_hbm, oden_hbm = passes[p]

        def _zero_rows(r, _):
            for j in range(HH // LANES):
                rows_v[r, pl.ds(j * LANES, LANES)] = zeros16
            return 0
        lax.fori_loop(0, CH, _zero_rows, 0)

        zb = jnp.zeros((LANES,), jnp.float32)
        def _zero_exbp(r, _):
            exb_v[r, pl.ds(4, 12)] = exb_v[r, pl.ds(4, 12)] if False else None
            return 0
        del _zero_exbp, zb

        for t in range(7):
            pltpu.sync_copy(rows_v, accn_s.at[pl.ds(base + t * CH, CH)])
        pltpu.sync_copy(rows_v.at[pl.ds(0, 64)],
                        accn_s.at[pl.ds(base + 560, 64)])

        def _zero_den(r, _):
            exb_v[r, pl.ds(0, LANES)] = zeros16
            return 0
        for t in range(7):
            pltpu.sync_copy(rows_v.at[pl.ds(0, CH), pl.ds(0, DW)],
                            accd_s.at[pl.ds(base + t * CH, CH)])
        pltpu.sync_copy(rows_v.at[pl.ds(0, 64), pl.ds(0, DW)],
                        accd_s.at[pl.ds(base + 560, 64)])

        @pl.when(s == NS - 1)
        def _():
            pltpu.sync_copy(rows_v.at[pl.ds(0, 16)],
                            accn_s.at[pl.ds(NN - 16, 16)])
            pltpu.sync_copy(rows_v.at[pl.ds(0, 16), pl.ds(0, DW)],
                            accd_s.at[pl.ds(NN - 16, 16)])

        plsc.subcore_barrier()

        m_splat = [m_v[2 * p + h, pl.ds(0, LANES)] for h in range(2)]

        def _gathers(jj, b):
            rw, ag, ad = b[0], b[1], b[2]
            sm = b[4]
            return (pltpu.make_async_copy(xw_hbm.at[src_v.at[jj]], rw, sm),
                    pltpu.make_async_copy(a16_hbm.at[src_v.at[jj]], ag, sm),
                    pltpu.make_async_copy(a16_hbm.at[dst_v.at[jj]], ad, sm))

        def _scatters(jj, b):
            rw, xb, sm = b[0], b[3], b[5]
            return (pltpu.make_async_copy(rw, accn_s.at[dst_v.at[jj]], sm),
                    pltpu.make_async_copy(xb, accd_s.at[dst_v.at[jj]], sm))

        for d in _gathers(0, bufs[0]):
            d.start()

        def _chunk_on(j, b, nb):
            rows_v, asrc_g, adst_g, exb_v = b[0], b[1], b[2], b[3]

            @pl.when(j >= 1)
            def _():
                for d in _scatters(j - 1, nb):
                    d.wait()

            @pl.when(j + 1 < NCHK)
            def _():
                for d in _gathers(j + 1, nb):
                    d.start()

            for d in _gathers(j, b):
                d.wait()

            for g in range(CH // LANES):
                e16i = g * LANES + iota16
                exg = []
                for h in range(2):
                    es = plsc.load_gather(
                        asrc_g, [e16i, jnp.full((LANES,), h + 1, jnp.int32)])
                    ed = plsc.load_gather(
                        adst_g, [e16i, jnp.full((LANES,), h + 3, jnp.int32)])
                    e = es + ed
                    e = jnp.where(e >= 0, e, e * 0.2)
                    ex = jnp.exp(e - m_splat[h])
                    exg.append(ex)
                    plsc.store_scatter(
                        exb_v,
                        [g * LANES + iota16,
                         jnp.full((LANES,), h + 1, jnp.int32)],
                        ex)
                for l in range(LANES):
                    e = g * LANES + l
                    li = jnp.full((LANES,), l, jnp.int32)
                    for h in range(2):
                        w = exg[h].at[li].get(mode="promise_in_bounds")
                        for jj in (2 * h, 2 * h + 1):
                            v = rows_v[e, pl.ds(jj * LANES, LANES)]
                            rows_v[e, pl.ds(jj * LANES, LANES)] = v * w

            for d in _scatters(j, b):
                d.start(add=True)

        def _chunk(j, _):
            @pl.when(j % 2 == 0)
            def _():
                _chunk_on(j, bufs[0], bufs[1])

            @pl.when(j % 2 == 1)
            def _():
                _chunk_on(j, bufs[1], bufs[0])
            return 0

        lax.fori_loop(0, NCHK, _chunk, 0)

        for d in _scatters(NCHK - 1, bufs[(NCHK - 1) % 2]):
            d.wait()

        plsc.subcore_barrier()

        pltpu.sync_copy(accn_s.at[pl.ds(base, SLAB)],
                        onum_hbm.at[c, pl.ds(base, SLAB)])
        pltpu.sync_copy(accd_s.at[pl.ds(base, SLAB)],
                        oden_hbm.at[c, pl.ds(base, SLAB)])

        @pl.when(s == NS - 1)
        def _():
            pltpu.sync_copy(accn_s.at[pl.ds(NN - 16, 16)],
                            onum_hbm.at[c, pl.ds(NN - 16, 16)])
            pltpu.sync_copy(accd_s.at[pl.ds(NN - 16, 16)],
                            oden_hbm.at[c, pl.ds(NN - 16, 16)])

        @pl.when(jnp.int32(p) == 0)
        def _():
            plsc.subcore_barrier()


def _gat_edges(src_r, dst_r, xw0, xw1, a0, a1, m0, m1):
    mesh = plsc.VectorSubcoreMesh(
        core_axis_name="c", subcore_axis_name="s",
        num_cores=NC, num_subcores=NS)
    f = pl.kernel(
        _gat_edge_body,
        out_type=(
            jax.ShapeDtypeStruct((NC, NN, HH), jnp.float32),
            jax.ShapeDtypeStruct((NC, NN, DW), jnp.float32),
            jax.ShapeDtypeStruct((NC, NN, HH), jnp.float32),
            jax.ShapeDtypeStruct((NC, NN, DW), jnp.float32),
        ),
        mesh=mesh,
        scratch_types=[
            pltpu.VMEM((NCHK, CH), jnp.int32),
            pltpu.VMEM((NCHK, CH), jnp.int32),
            pltpu.VMEM((CH, 16), jnp.float32),
            pltpu.VMEM((CH, 16), jnp.float32),
            pltpu.VMEM((4, LANES), jnp.float32),
            pltpu.VMEM((CH, HH), jnp.float32),
            pltpu.VMEM((CH, DW), jnp.float32),
            pltpu.VMEM((CH, HH), jnp.float32),
            pltpu.VMEM((CH, 16), jnp.float32),
            pltpu.VMEM((CH, 16), jnp.float32),
            pltpu.VMEM((CH, DW), jnp.float32),
            pltpu.VMEM_SHARED((NN, HH), jnp.float32),
            pltpu.VMEM_SHARED((NN, DW), jnp.float32),
            pltpu.SemaphoreType.DMA,
            pltpu.SemaphoreType.DMA,
            pltpu.SemaphoreType.DMA,
            pltpu.SemaphoreType.DMA,
        ],
        compiler_params=pltpu.CompilerParams(
            needs_layout_passes=False, use_tc_tiling_on_sc=False),
    )
    return f(src_r, dst_r, xw0, xw1, a0, a1, m0, m1)


NB = 10
BR = NN // NB


def _b2a_body(pn0_ref, pn1_ref, pd0_ref, pd1_ref, se_ref, xw0_ref, xw1_ref,
              gb_ref, hpre_ref, dinv_ref, s1_ref, s2_ref):
    hsel = (lax.broadcasted_iota(jnp.int32, (NH, HID1), 1) // CC ==
            lax.broadcasted_iota(jnp.int32, (NH, HID1), 0)).astype(jnp.float32)
    exs = se_ref[...][:, 0:4]
    xw = jnp.concatenate([xw0_ref[...], xw1_ref[...]], axis=1)
    num = (jnp.concatenate([pn0_ref[0] + pn0_ref[1],
                            pn1_ref[0] + pn1_ref[1]], axis=1) +
           jnp.dot(exs, hsel, preferred_element_type=jnp.float32) * xw)
    den4 = jnp.concatenate(
        [pd0_ref[0][:, 1:3] + pd0_ref[1][:, 1:3],
         pd1_ref[0][:, 1:3] + pd1_ref[1][:, 1:3]], axis=1) + exs
    deg = pd0_ref[0][:, 3:4] + pd0_ref[1][:, 3:4] + 1.0
    den = jnp.dot(den4, hsel, preferred_element_type=jnp.float32) + 1e-16
    h = num / den + gb_ref[...]
    h = jnp.where(h > 0, h, jnp.exp(jnp.minimum(h, 0.0)) - 1.0)
    hpre_ref[...] = h
    dinv_ref[...] = jnp.where(deg > 0, 1.0 / jnp.sqrt(deg), 0.0)

    @pl.when(pl.program_id(0) == 0)
    def _():
        s1_ref[...] = jnp.zeros_like(s1_ref)
        s2_ref[...] = jnp.zeros_like(s2_ref)
    s1_ref[...] += jnp.sum(h, axis=0, keepdims=True)
    s2_ref[...] += jnp.sum(h * h, axis=0, keepdims=True)


def _b2b_body(hpre_ref, dinv_ref, s1_ref, s2_ref, g1_ref, b1_ref, gw_ref,
              hw_ref, h2s_ref):
    mean = s1_ref[...] / NN
    var = s2_ref[...] / NN - mean * mean
    h = (g1_ref[...] * (hpre_ref[...] - mean) / jnp.sqrt(var + 1e-5) +
         b1_ref[...])
    hw = jnp.dot(h, gw_ref[...], preferred_element_type=jnp.float32)
    hw_ref[...] = hw
    dinv = dinv_ref[...]
    h2s_ref[...] = hw * (dinv * dinv)


def _stage_b2(pn0, pn1, pd0, pd1, selfex, xw0, xw1,
              gat_bias, bn1_g, bn1_b, gcn_w):
    bs_pn = pl.BlockSpec((NC, BR, HH), lambda i: (0, i, 0))
    bs_pd = pl.BlockSpec((NC, BR, DW), lambda i: (0, i, 0))
    bs_full128 = pl.BlockSpec((1, HID1), lambda i: (0, 0))
    hpre, dinv2d, s1, s2 = pl.pallas_call(
        _b2a_body,
        grid=(NB,),
        in_specs=[
            bs_pn, bs_pn, bs_pd, bs_pd,
            pl.BlockSpec((BR, 8), lambda i: (i, 0)),
            pl.BlockSpec((BR, HH), lambda i: (i, 0)),
            pl.BlockSpec((BR, HH), lambda i: (i, 0)),
            bs_full128,
        ],
        out_specs=[
            pl.BlockSpec((BR, HID1), lambda i: (i, 0)),
            pl.BlockSpec((BR, 1), lambda i: (i, 0)),
            bs_full128,
            bs_full128,
        ],
        out_shape=(
            jax.ShapeDtypeStruct((NN, HID1), jnp.float32),
            jax.ShapeDtypeStruct((NN, 1), jnp.float32),
            jax.ShapeDtypeStruct((1, HID1), jnp.float32),
            jax.ShapeDtypeStruct((1, HID1), jnp.float32),
        ),
    )(pn0, pn1, pd0, pd1, selfex, xw0, xw1, gat_bias)

    hw, h2s = pl.pallas_call(
        _b2b_body,
        grid=(NB,),
        in_specs=[
            pl.BlockSpec((BR, HID1), lambda i: (i, 0)),
            pl.BlockSpec((BR, 1), lambda i: (i, 0)),
            bs_full128, bs_full128, bs_full128, bs_full128,
            pl.BlockSpec((HID1, HID2), lambda i: (0, 0)),
        ],
        out_specs=[
            pl.BlockSpec((BR, HID2), lambda i: (i, 0)),
            pl.BlockSpec((BR, HID2), lambda i: (i, 0)),
        ],
        out_shape=(
            jax.ShapeDtypeStruct((NN, HID2), jnp.float32),
            jax.ShapeDtypeStruct((NN, HID2), jnp.float32),
        ),
    )(hpre, dinv2d, s1, s2, bn1_g, bn1_b, gcn_w)
    return hw, dinv2d, h2s


def _gcn_edge_body(src_hbm, dst_hbm, hw_hbm, dinv_hbm, out_hbm,
                   src_v, dst_v, dinv_v, rows_v, rows_w, acc_s,
                   sem, sem2, ssem, ssem2):
    c = lax.axis_index("c")
    s = lax.axis_index("s")
    wid = c * NS + s
    base = s * SLAB

    pltpu.sync_copy(src_hbm.at[wid], src_v)
    pltpu.sync_copy(dst_hbm.at[wid], dst_v)
    pltpu.sync_copy(dinv_hbm, dinv_v)

    zeros16 = jnp.zeros((LANES,), jnp.float32)

    def _zero_rows(r, _):
        for j in range(HID2 // LANES):
            rows_v[r, pl.ds(j * LANES, LANES)] = zeros16
        return 0
    lax.fori_loop(0, CH, _zero_rows, 0)

    for t in range(7):
        pltpu.sync_copy(rows_v, acc_s.at[pl.ds(base + t * CH, CH)])
    pltpu.sync_copy(rows_v.at[pl.ds(0, 64)], acc_s.at[pl.ds(base + 560, 64)])

    @pl.when(s == NS - 1)
    def _():
        pltpu.sync_copy(rows_v.at[pl.ds(0, 16)], acc_s.at[pl.ds(NN - 16, 16)])

    plsc.subcore_barrier()

    gbufs = ((rows_v, sem, ssem), (rows_w, sem2, ssem2))

    def _scat(jj, b):
        return pltpu.make_async_copy(b[0], acc_s.at[dst_v.at[jj]], b[2])

    pltpu.make_async_copy(hw_hbm.at[src_v.at[0]], rows_v, sem).start()

    def _chunk_on(j, b, nb):
        rows_v, sm = b[0], b[1]

        @pl.when(j >= 1)
        def _():
            _scat(j - 1, nb).wait()

        @pl.when(j + 1 < NCHK)
        def _():
            pltpu.make_async_copy(hw_hbm.at[src_v.at[j + 1]], nb[0],
                                  nb[1]).start()

        pltpu.make_async_copy(hw_hbm.at[src_v.at[j]], rows_v, sm).wait()

        for g in range(CH // LANES):
            src16 = src_v[j, pl.ds(g * LANES, LANES)]
            dst16 = dst_v[j, pl.ds(g * LANES, LANES)]
            nv = (plsc.load_gather(dinv_v, [src16]) *
                  plsc.load_gather(dinv_v, [dst16]))
            for l in range(LANES):
                e = g * LANES + l
                w = nv.at[jnp.full((LANES,), l, jnp.int32)].get(
                    mode="promise_in_bounds")
                for jj in range(HID2 // LANES):
                    v = rows_v[e, pl.ds(jj * LANES, LANES)]
                    rows_v[e, pl.ds(jj * LANES, LANES)] = v * w

        _scat(j, b).start(add=True)

    def _chunk(j, _):
        @pl.when(j % 2 == 0)
        def _():
            _chunk_on(j, gbufs[0], gbufs[1])

        @pl.when(j % 2 == 1)
        def _():
            _chunk_on(j, gbufs[1], gbufs[0])
        return 0

    lax.fori_loop(0, NCHK, _chunk, 0)

    _scat(NCHK - 1, gbufs[(NCHK - 1) % 2]).wait()

    plsc.subcore_barrier()

    pltpu.sync_copy(acc_s.at[pl.ds(base, SLAB)],
                    out_hbm.at[c, pl.ds(base, SLAB)])

    @pl.when(s == NS - 1)
    def _():
        pltpu.sync_copy(acc_s.at[pl.ds(NN - 16, 16)],
                        out_hbm.at[c, pl.ds(NN - 16, 16)])


def _gcn_edges(src_r, dst_r, hw, dinv):
    mesh = plsc.VectorSubcoreMesh(
        core_axis_name="c", subcore_axis_name="s",
        num_cores=NC, num_subcores=NS)
    f = pl.kernel(
        _gcn_edge_body,
        out_type=jax.ShapeDtypeStruct((NC, NN, HID2), jnp.float32),
        mesh=mesh,
        scratch_types=[
            pltpu.VMEM((NCHK, CH), jnp.int32),
            pltpu.VMEM((NCHK, CH), jnp.int32),
            pltpu.VMEM((NN,), jnp.float32),
            pltpu.VMEM((CH, HID2), jnp.float32),
            pltpu.VMEM((CH, HID2), jnp.float32),
            pltpu.VMEM_SHARED((NN, HID2), jnp.float32),
            pltpu.SemaphoreType.DMA,
            pltpu.SemaphoreType.DMA,
            pltpu.SemaphoreType.DMA,
            pltpu.SemaphoreType.DMA,
        ],
        compiler_params=pltpu.CompilerParams(
            needs_layout_passes=False, use_tc_tiling_on_sc=False),
    )
    return f(src_r, dst_r, hw, dinv)


def _d_body(p2_ref, h2s_ref, gcb_ref, g2_ref, b2_ref, gw_ref, gb_ref,
            fw_ref, fb_ref, batch_ref, out_ref):
    h2 = p2_ref[0] + p2_ref[1] + h2s_ref[...] + gcb_ref[...]
    h2 = jnp.where(h2 > 0, h2, jnp.exp(jnp.minimum(h2, 0.0)) - 1.0)
    mean = jnp.mean(h2, axis=0, keepdims=True)
    var = jnp.mean((h2 - mean) * (h2 - mean), axis=0, keepdims=True)
    h2 = g2_ref[...] * (h2 - mean) / jnp.sqrt(var + 1e-5) + b2_ref[...]
    gate = jnp.dot(h2, gw_ref[...], preferred_element_type=jnp.float32)
    gate = gate + gb_ref[...]
    gmax = jnp.max(gate)
    ge = jnp.exp(gate - gmax)
    onehot = (batch_ref[...] ==
              lax.broadcasted_iota(jnp.int32, (NN, NG), 1)).astype(jnp.float32)
    gden = lax.dot_general(onehot, ge, (((0,), (0,)), ((), ())),
                           preferred_element_type=jnp.float32)
    gden_n = jnp.dot(onehot, gden, preferred_element_type=jnp.float32)
    attn = ge / (gden_n + 1e-16)
    rep = lax.dot_general(onehot, attn * h2, (((0,), (0,)), ((), ())),
                          preferred_element_type=jnp.float32)
    out = jnp.dot(rep, fw_ref[...], preferred_element_type=jnp.float32)
    out_ref[...] = out + fb_ref[...]


def _stage_d(p2, h2s, gcn_bias, bn2_g, bn2_b, gate_w, gate_b, fc_w, fc_b,
             batch2d):
    return pl.pallas_call(
        _d_body,
        out_shape=jax.ShapeDtypeStruct((NG, 1), jnp.float32),
    )(p2, h2s, gcn_bias, bn2_g, bn2_b, gate_w, gate_b, fc_w, fc_b, batch2d)


def kernel(x, edge_index, batch, gat_W, gat_att_src, gat_att_dst, gat_bias,
           bn1_gamma, bn1_beta, gcn_W, gcn_bias, bn2_gamma, bn2_beta,
           gate_W, gate_b, fc_W, fc_b):
    hsel = ((jnp.arange(HID1, dtype=jnp.int32)[:, None] // CC) ==
            jnp.arange(NH, dtype=jnp.int32)[None, :])
    a_src_m = jnp.where(hsel, gat_att_src.reshape(HID1)[:, None], 0.0)
    a_dst_m = jnp.where(hsel, gat_att_dst.reshape(HID1)[:, None], 0.0)
    ab = jnp.concatenate([a_src_m, a_dst_m], axis=1).astype(jnp.float32)

    xw0, xw1, a4_0, a4_1, selfex, m16_0, m16_1 = _stage_a(x, gat_W, ab)

    src_r = edge_index[0].reshape(NW, NCHK, CH)
    dst_r = edge_index[1].reshape(NW, NCHK, CH)

    pn0, pd0, pn1, pd1 = _gat_edges(
        src_r, dst_r, xw0, xw1, a4_0, a4_1, m16_0, m16_1)

    hw, dinv2d, h2s = _stage_b2(
        pn0, pn1, pd0, pd1, selfex, xw0, xw1,
        gat_bias.reshape(1, HID1), bn1_gamma.reshape(1, HID1),
        bn1_beta.reshape(1, HID1), gcn_W)

    p2 = _gcn_edges(src_r, dst_r, hw, dinv2d.reshape(NN))

    out = _stage_d(
        p2, h2s, gcn_bias.reshape(1, HID2), bn2_gamma.reshape(1, HID2),
        bn2_beta.reshape(1, HID2), gate_W, gate_b.reshape(1, 1),
        fc_W, fc_b.reshape(1, 1), batch.reshape(NN, 1))

    return out.reshape(NG)


# speedup vs baseline: 1.0346x; 1.0346x over previous
import jax
import jax.numpy as jnp
from jax import lax
from jax.experimental import pallas as pl
from jax.experimental.pallas import tpu as pltpu
from jax.experimental.pallas import tpu_sc as plsc

NN = 10000
EE = 320000
NH = 4
CC = 32
HID1 = NH * CC
HH = HID1 // 2
HID2 = 64
NG = 64

NC = 2
NS = 16
LANES = 16
NW = NC * NS
EPT = EE // NW
CH = 80
NCHK = EPT // CH
SLAB = 624
DW = 16


def _a_body(x_ref, w_ref, ab_ref, xw0_ref, xw1_ref, a16_0_ref, a16_1_ref,
            se_ref, m0_ref, m1_ref):
    xw = jnp.dot(x_ref[...], w_ref[...], preferred_element_type=jnp.float32)
    xw0_ref[...] = xw[:, 0:HH]
    xw1_ref[...] = xw[:, HH:HID1]
    a8 = jnp.dot(xw, ab_ref[...], preferred_element_type=jnp.float32)
    z1 = jnp.zeros((NN, 1), jnp.float32)
    z11 = jnp.zeros((NN, 11), jnp.float32)
    a16_0_ref[...] = jnp.concatenate(
        [z1, a8[:, 0:2], a8[:, 4:6], z11], axis=1)
    a16_1_ref[...] = jnp.concatenate(
        [z1, a8[:, 2:4], a8[:, 6:8], z11], axis=1)
    amax = jnp.max(a8, axis=0, keepdims=True)
    m = amax[:, 0:4] + amax[:, 4:8]
    es = a8[:, 0:4] + a8[:, 4:8]
    es = jnp.where(es >= 0, es, es * 0.2)
    exs = jnp.exp(es - m)
    se_ref[...] = jnp.concatenate([exs, jnp.zeros_like(exs)], axis=1)
    ml = jnp.tile(m.reshape(4, 1), (1, LANES))
    m0_ref[...] = ml[0:2]
    m1_ref[...] = ml[2:4]


def _stage_a(x, gat_w, ab):
    return pl.pallas_call(
        _a_body,
        out_shape=(
            jax.ShapeDtypeStruct((NN, HH), jnp.float32),
            jax.ShapeDtypeStruct((NN, HH), jnp.float32),
            jax.ShapeDtypeStruct((NN, 16), jnp.float32),
            jax.ShapeDtypeStruct((NN, 16), jnp.float32),
            jax.ShapeDtypeStruct((NN, 8), jnp.float32),
            jax.ShapeDtypeStruct((2, LANES), jnp.float32),
            jax.ShapeDtypeStruct((2, LANES), jnp.float32),
        ),
    )(x, gat_w, ab)


def _gat_edge_body(src_hbm, dst_hbm, xw_hbm, a16_hbm, m_hbm,
                   onum_hbm, oden_hbm,
                   src_v, dst_v, asrc_g, adst_g, m_v, rows_v, exb_v,
                   rows_w, asrc_h, adst_h, exb_w,
                   accn_s, accd_s, sem, sem2, ssem, ssem2):
    c = lax.axis_index("c")
    s = lax.axis_index("s")
    wid = c * NS + s
    base = s * SLAB

    pltpu.sync_copy(src_hbm.at[wid], src_v)
    pltpu.sync_copy(dst_hbm.at[wid], dst_v)
    pltpu.sync_copy(m_hbm, m_v)

    zeros16 = jnp.zeros((LANES,), jnp.float32)

    def _zero_rows(r, _):
        for j in range(HH // LANES):
            rows_v[r, pl.ds(j * LANES, LANES)] = zeros16
        exb_v[r, pl.ds(0, LANES)] = zeros16
        exb_w[r, pl.ds(0, LANES)] = zeros16
        return 0
    lax.fori_loop(0, CH, _zero_rows, 0)

    for t in range(7):
        pltpu.sync_copy(rows_v, accn_s.at[pl.ds(base + t * CH, CH)])
        pltpu.sync_copy(exb_v, accd_s.at[pl.ds(base + t * CH, CH)])
    pltpu.sync_copy(rows_v.at[pl.ds(0, 64)], accn_s.at[pl.ds(base + 560, 64)])
    pltpu.sync_copy(exb_v.at[pl.ds(0, 64)], accd_s.at[pl.ds(base + 560, 64)])

    @pl.when(s == NS - 1)
    def _():
        pltpu.sync_copy(rows_v.at[pl.ds(0, 16)],
                        accn_s.at[pl.ds(NN - 16, 16)])
        pltpu.sync_copy(exb_v.at[pl.ds(0, 16)],
                        accd_s.at[pl.ds(NN - 16, 16)])

    iota16 = lax.iota(jnp.int32, LANES)
    ones16 = jnp.ones((LANES,), jnp.float32)
    for g in range(CH // LANES):
        for xb in (exb_v, exb_w):
            plsc.store_scatter(
                xb, [g * LANES + iota16, jnp.full((LANES,), 3, jnp.int32)],
                ones16)

    plsc.subcore_barrier()

    m_splat = [m_v[h, pl.ds(0, LANES)] for h in range(2)]

    bufs = ((rows_v, asrc_g, adst_g, exb_v, sem, ssem),
            (rows_w, asrc_h, adst_h, exb_w, sem2, ssem2))

    def _gathers(jj, b):
        rw, ag, ad = b[0], b[1], b[2]
        sm = b[4]
        return (pltpu.make_async_copy(xw_hbm.at[src_v.at[jj]], rw, sm),
                pltpu.make_async_copy(a16_hbm.at[src_v.at[jj]], ag, sm),
                pltpu.make_async_copy(a16_hbm.at[dst_v.at[jj]], ad, sm))

    def _scatters(jj, b):
        rw, xb, sm = b[0], b[3], b[5]
        return (pltpu.make_async_copy(rw, accn_s.at[dst_v.at[jj]], sm),
                pltpu.make_async_copy(xb, accd_s.at[dst_v.at[jj]], sm))

    for d in _gathers(0, bufs[0]):
        d.start()

    def _chunk_on(j, b, nb):
        rows_v, asrc_g, adst_g, exb_v = b[0], b[1], b[2], b[3]

        @pl.when(j >= 1)
        def _():
            for d in _scatters(j - 1, nb):
                d.wait()

        @pl.when(j + 1 < NCHK)
        def _():
            for d in _gathers(j + 1, nb):
                d.start()

        for d in _gathers(j, b):
            d.wait()

        for g in range(CH // LANES):
            e16i = g * LANES + iota16
            exg = []
            for h in range(2):
                es = plsc.load_gather(
                    asrc_g, [e16i, jnp.full((LANES,), h + 1, jnp.int32)])
                ed = plsc.load_gather(
                    adst_g, [e16i, jnp.full((LANES,), h + 3, jnp.int32)])
                e = es + ed
                e = jnp.where(e >= 0, e, e * 0.2)
                ex = jnp.exp(e - m_splat[h])
                exg.append(ex)
                plsc.store_scatter(
                    exb_v,
                    [g * LANES + iota16, jnp.full((LANES,), h + 1, jnp.int32)],
                    ex)
            for l in range(LANES):
                e = g * LANES + l
                li = jnp.full((LANES,), l, jnp.int32)
                for h in range(2):
                    w = exg[h].at[li].get(mode="promise_in_bounds")
                    for jj in (2 * h, 2 * h + 1):
                        v = rows_v[e, pl.ds(jj * LANES, LANES)]
                        rows_v[e, pl.ds(jj * LANES, LANES)] = v * w

        for d in _scatters(j, b):
            d.start(add=True)

    def _chunk(j, _):
        @pl.when(j % 2 == 0)
        def _():
            _chunk_on(j, bufs[0], bufs[1])

        @pl.when(j % 2 == 1)
        def _():
            _chunk_on(j, bufs[1], bufs[0])
        return 0

    lax.fori_loop(0, NCHK, _chunk, 0)

    for d in _scatters(NCHK - 1, bufs[(NCHK - 1) % 2]):
        d.wait()

    plsc.subcore_barrier()

    pltpu.sync_copy(accn_s.at[pl.ds(base, SLAB)],
                    onum_hbm.at[c, pl.ds(base, SLAB)])
    pltpu.sync_copy(accd_s.at[pl.ds(base, SLAB)],
                    oden_hbm.at[c, pl.ds(base, SLAB)])

    @pl.when(s == NS - 1)
    def _():
        pltpu.sync_copy(accn_s.at[pl.ds(NN - 16, 16)],
                        onum_hbm.at[c, pl.ds(NN - 16, 16)])
        pltpu.sync_copy(accd_s.at[pl.ds(NN - 16, 16)],
                        oden_hbm.at[c, pl.ds(NN - 16, 16)])


def _gat_edges(src_r, dst_r, xwh, a16, m16):
    mesh = plsc.VectorSubcoreMesh(
        core_axis_name="c", subcore_axis_name="s",
        num_cores=NC, num_subcores=NS)
    f = pl.kernel(
        _gat_edge_body,
        out_type=(
            jax.ShapeDtypeStruct((NC, NN, HH), jnp.float32),
            jax.ShapeDtypeStruct((NC, NN, DW), jnp.float32),
        ),
        mesh=mesh,
        scratch_types=[
            pltpu.VMEM((NCHK, CH), jnp.int32),
            pltpu.VMEM((NCHK, CH), jnp.int32),
            pltpu.VMEM((CH, 16), jnp.float32),
            pltpu.VMEM((CH, 16), jnp.float32),
            pltpu.VMEM((2, LANES), jnp.float32),
            pltpu.VMEM((CH, HH), jnp.float32),
            pltpu.VMEM((CH, DW), jnp.float32),
            pltpu.VMEM((CH, HH), jnp.float32),
            pltpu.VMEM((CH, 16), jnp.float32),
            pltpu.VMEM((CH, 16), jnp.float32),
            pltpu.VMEM((CH, DW), jnp.float32),
            pltpu.VMEM_SHARED((NN, HH), jnp.float32),
            pltpu.VMEM_SHARED((NN, DW), jnp.float32),
            pltpu.SemaphoreType.DMA,
            pltpu.SemaphoreType.DMA,
            pltpu.SemaphoreType.DMA,
            pltpu.SemaphoreType.DMA,
        ],
        compiler_params=pltpu.CompilerParams(
            needs_layout_passes=False, use_tc_tiling_on_sc=False),
    )
    return f(src_r, dst_r, xwh, a16, m16)


NB = 10
BR = NN // NB


def _b2a_body(pn0_ref, pn1_ref, pd0_ref, pd1_ref, se_ref, xw0_ref, xw1_ref,
              gb_ref, hpre_ref, dinv_ref, s1_ref, s2_ref):
    hsel = (lax.broadcasted_iota(jnp.int32, (NH, HID1), 1) // CC ==
            lax.broadcasted_iota(jnp.int32, (NH, HID1), 0)).astype(jnp.float32)
    exs = se_ref[...][:, 0:4]
    xw = jnp.concatenate([xw0_ref[...], xw1_ref[...]], axis=1)
    num = (jnp.concatenate([pn0_ref[0] + pn0_ref[1],
                            pn1_ref[0] + pn1_ref[1]], axis=1) +
           jnp.dot(exs, hsel, preferred_element_type=jnp.float32) * xw)
    den4 = jnp.concatenate(
        [pd0_ref[0][:, 1:3] + pd0_ref[1][:, 1:3],
         pd1_ref[0][:, 1:3] + pd1_ref[1][:, 1:3]], axis=1) + exs
    deg = pd0_ref[0][:, 3:4] + pd0_ref[1][:, 3:4] + 1.0
    den = jnp.dot(den4, hsel, preferred_element_type=jnp.float32) + 1e-16
    h = num / den + gb_ref[...]
    h = jnp.where(h > 0, h, jnp.exp(jnp.minimum(h, 0.0)) - 1.0)
    hpre_ref[...] = h
    dinv_ref[...] = jnp.where(deg > 0, 1.0 / jnp.sqrt(deg), 0.0)

    @pl.when(pl.program_id(0) == 0)
    def _():
        s1_ref[...] = jnp.zeros_like(s1_ref)
        s2_ref[...] = jnp.zeros_like(s2_ref)
    s1_ref[...] += jnp.sum(h, axis=0, keepdims=True)
    s2_ref[...] += jnp.sum(h * h, axis=0, keepdims=True)


def _b2b_body(hpre_ref, dinv_ref, s1_ref, s2_ref, g1_ref, b1_ref, gw_ref,
              hw_ref, h2s_ref):
    mean = s1_ref[...] / NN
    var = s2_ref[...] / NN - mean * mean
    h = (g1_ref[...] * (hpre_ref[...] - mean) / jnp.sqrt(var + 1e-5) +
         b1_ref[...])
    hw = jnp.dot(h, gw_ref[...], preferred_element_type=jnp.float32)
    hw_ref[...] = hw
    dinv = dinv_ref[...]
    h2s_ref[...] = hw * (dinv * dinv)


def _stage_b2(pn0, pn1, pd0, pd1, selfex, xw0, xw1,
              gat_bias, bn1_g, bn1_b, gcn_w):
    bs_pn = pl.BlockSpec((NC, BR, HH), lambda i: (0, i, 0))
    bs_pd = pl.BlockSpec((NC, BR, DW), lambda i: (0, i, 0))
    bs_full128 = pl.BlockSpec((1, HID1), lambda i: (0, 0))
    hpre, dinv2d, s1, s2 = pl.pallas_call(
        _b2a_body,
        grid=(NB,),
        in_specs=[
            bs_pn, bs_pn, bs_pd, bs_pd,
            pl.BlockSpec((BR, 8), lambda i: (i, 0)),
            pl.BlockSpec((BR, HH), lambda i: (i, 0)),
            pl.BlockSpec((BR, HH), lambda i: (i, 0)),
            bs_full128,
        ],
        out_specs=[
            pl.BlockSpec((BR, HID1), lambda i: (i, 0)),
            pl.BlockSpec((BR, 1), lambda i: (i, 0)),
            bs_full128,
            bs_full128,
        ],
        out_shape=(
            jax.ShapeDtypeStruct((NN, HID1), jnp.float32),
            jax.ShapeDtypeStruct((NN, 1), jnp.float32),
            jax.ShapeDtypeStruct((1, HID1), jnp.float32),
            jax.ShapeDtypeStruct((1, HID1), jnp.float32),
        ),
    )(pn0, pn1, pd0, pd1, selfex, xw0, xw1, gat_bias)

    hw, h2s = pl.pallas_call(
        _b2b_body,
        grid=(NB,),
        in_specs=[
            pl.BlockSpec((BR, HID1), lambda i: (i, 0)),
            pl.BlockSpec((BR, 1), lambda i: (i, 0)),
            bs_full128, bs_full128, bs_full128, bs_full128,
            pl.BlockSpec((HID1, HID2), lambda i: (0, 0)),
        ],
        out_specs=[
            pl.BlockSpec((BR, HID2), lambda i: (i, 0)),
            pl.BlockSpec((BR, HID2), lambda i: (i, 0)),
        ],
        out_shape=(
            jax.ShapeDtypeStruct((NN, HID2), jnp.float32),
            jax.ShapeDtypeStruct((NN, HID2), jnp.float32),
        ),
    )(hpre, dinv2d, s1, s2, bn1_g, bn1_b, gcn_w)
    return hw, dinv2d, h2s


def _gcn_edge_body(src_hbm, dst_hbm, hw_hbm, dinv_hbm, out_hbm,
                   src_v, dst_v, dinv_v, rows_v, rows_w, acc_s,
                   sem, sem2, ssem, ssem2):
    c = lax.axis_index("c")
    s = lax.axis_index("s")
    wid = c * NS + s
    base = s * SLAB

    pltpu.sync_copy(src_hbm.at[wid], src_v)
    pltpu.sync_copy(dst_hbm.at[wid], dst_v)
    pltpu.sync_copy(dinv_hbm, dinv_v)

    zeros16 = jnp.zeros((LANES,), jnp.float32)

    def _zero_rows(r, _):
        for j in range(HID2 // LANES):
            rows_v[r, pl.ds(j * LANES, LANES)] = zeros16
        return 0
    lax.fori_loop(0, CH, _zero_rows, 0)

    for t in range(7):
        pltpu.sync_copy(rows_v, acc_s.at[pl.ds(base + t * CH, CH)])
    pltpu.sync_copy(rows_v.at[pl.ds(0, 64)], acc_s.at[pl.ds(base + 560, 64)])

    @pl.when(s == NS - 1)
    def _():
        pltpu.sync_copy(rows_v.at[pl.ds(0, 16)], acc_s.at[pl.ds(NN - 16, 16)])

    plsc.subcore_barrier()

    gbufs = ((rows_v, sem, ssem), (rows_w, sem2, ssem2))

    def _scat(jj, b):
        return pltpu.make_async_copy(b[0], acc_s.at[dst_v.at[jj]], b[2])

    pltpu.make_async_copy(hw_hbm.at[src_v.at[0]], rows_v, sem).start()

    def _chunk_on(j, b, nb):
        rows_v, sm = b[0], b[1]

        @pl.when(j >= 1)
        def _():
            _scat(j - 1, nb).wait()

        @pl.when(j + 1 < NCHK)
        def _():
            pltpu.make_async_copy(hw_hbm.at[src_v.at[j + 1]], nb[0],
                                  nb[1]).start()

        pltpu.make_async_copy(hw_hbm.at[src_v.at[j]], rows_v, sm).wait()

        for g in range(CH // LANES):
            src16 = src_v[j, pl.ds(g * LANES, LANES)]
            dst16 = dst_v[j, pl.ds(g * LANES, LANES)]
            nv = (plsc.load_gather(dinv_v, [src16]) *
                  plsc.load_gather(dinv_v, [dst16]))
            for l in range(LANES):
                e = g * LANES + l
                w = nv.at[jnp.full((LANES,), l, jnp.int32)].get(
                    mode="promise_in_bounds")
                for jj in range(HID2 // LANES):
                    v = rows_v[e, pl.ds(jj * LANES, LANES)]
                    rows_v[e, pl.ds(jj * LANES, LANES)] = v * w

        _scat(j, b).start(add=True)

    def _chunk(j, _):
        @pl.when(j % 2 == 0)
        def _():
            _chunk_on(j, gbufs[0], gbufs[1])

        @pl.when(j % 2 == 1)
        def _():
            _chunk_on(j, gbufs[1], gbufs[0])
        return 0

    lax.fori_loop(0, NCHK, _chunk, 0)

    _scat(NCHK - 1, gbufs[(NCHK - 1) % 2]).wait()

    plsc.subcore_barrier()

    pltpu.sync_copy(acc_s.at[pl.ds(base, SLAB)],
                    out_hbm.at[c, pl.ds(base, SLAB)])

    @pl.when(s == NS - 1)
    def _():
        pltpu.sync_copy(acc_s.at[pl.ds(NN - 16, 16)],
                        out_hbm.at[c, pl.ds(NN - 16, 16)])


def _gcn_edges(src_r, dst_r, hw, dinv):
    mesh = plsc.VectorSubcoreMesh(
        core_axis_name="c", subcore_axis_name="s",
        num_cores=NC, num_subcores=NS)
    f = pl.kernel(
        _gcn_edge_body,
        out_type=jax.ShapeDtypeStruct((NC, NN, HID2), jnp.float32),
        mesh=mesh,
        scratch_types=[
            pltpu.VMEM((NCHK, CH), jnp.int32),
            pltpu.VMEM((NCHK, CH), jnp.int32),
            pltpu.VMEM((NN,), jnp.float32),
            pltpu.VMEM((CH, HID2), jnp.float32),
            pltpu.VMEM((CH, HID2), jnp.float32),
            pltpu.VMEM_SHARED((NN, HID2), jnp.float32),
            pltpu.SemaphoreType.DMA,
            pltpu.SemaphoreType.DMA,
            pltpu.SemaphoreType.DMA,
            pltpu.SemaphoreType.DMA,
        ],
        compiler_params=pltpu.CompilerParams(
            needs_layout_passes=False, use_tc_tiling_on_sc=False),
    )
    return f(src_r, dst_r, hw, dinv)


def _d_body(p2_ref, h2s_ref, gcb_ref, g2_ref, b2_ref, gw_ref, gb_ref,
            fw_ref, fb_ref, batch_ref, out_ref):
    h2 = p2_ref[0] + p2_ref[1] + h2s_ref[...] + gcb_ref[...]
    h2 = jnp.where(h2 > 0, h2, jnp.exp(jnp.minimum(h2, 0.0)) - 1.0)
    mean = jnp.mean(h2, axis=0, keepdims=True)
    var = jnp.mean((h2 - mean) * (h2 - mean), axis=0, keepdims=True)
    h2 = g2_ref[...] * (h2 - mean) / jnp.sqrt(var + 1e-5) + b2_ref[...]
    gate = jnp.dot(h2, gw_ref[...], preferred_element_type=jnp.float32)
    gate = gate + gb_ref[...]
    gmax = jnp.max(gate)
    ge = jnp.exp(gate - gmax)
    onehot = (batch_ref[...] ==
              lax.broadcasted_iota(jnp.int32, (NN, NG), 1)).astype(jnp.float32)
    gden = lax.dot_general(onehot, ge, (((0,), (0,)), ((), ())),
                           preferred_element_type=jnp.float32)
    gden_n = jnp.dot(onehot, gden, preferred_element_type=jnp.float32)
    attn = ge / (gden_n + 1e-16)
    rep = lax.dot_general(onehot, attn * h2, (((0,), (0,)), ((), ())),
                          preferred_element_type=jnp.float32)
    out = jnp.dot(rep, fw_ref[...], preferred_element_type=jnp.float32)
    out_ref[...] = out + fb_ref[...]


def _stage_d(p2, h2s, gcn_bias, bn2_g, bn2_b, gate_w, gate_b, fc_w, fc_b,
             batch2d):
    return pl.pallas_call(
        _d_body,
        out_shape=jax.ShapeDtypeStruct((NG, 1), jnp.float32),
    )(p2, h2s, gcn_bias, bn2_g, bn2_b, gate_w, gate_b, fc_w, fc_b, batch2d)


def kernel(x, edge_index, batch, gat_W, gat_att_src, gat_att_dst, gat_bias,
           bn1_gamma, bn1_beta, gcn_W, gcn_bias, bn2_gamma, bn2_beta,
           gate_W, gate_b, fc_W, fc_b):
    hsel = ((jnp.arange(HID1, dtype=jnp.int32)[:, None] // CC) ==
            jnp.arange(NH, dtype=jnp.int32)[None, :])
    a_src_m = jnp.where(hsel, gat_att_src.reshape(HID1)[:, None], 0.0)
    a_dst_m = jnp.where(hsel, gat_att_dst.reshape(HID1)[:, None], 0.0)
    ab = jnp.concatenate([a_src_m, a_dst_m], axis=1).astype(jnp.float32)

    xw0, xw1, a4_0, a4_1, selfex, m16_0, m16_1 = _stage_a(x, gat_W, ab)

    src_r = edge_index[0].reshape(NW, NCHK, CH)
    dst_r = edge_index[1].reshape(NW, NCHK, CH)

    pn0, pd0 = _gat_edges(src_r, dst_r, xw0, a4_0, m16_0)
    pn1, pd1 = _gat_edges(src_r, dst_r, xw1, a4_1, m16_1)

    hw, dinv2d, h2s = _stage_b2(
        pn0, pn1, pd0, pd1, selfex, xw0, xw1,
        gat_bias.reshape(1, HID1), bn1_gamma.reshape(1, HID1),
        bn1_beta.reshape(1, HID1), gcn_W)

    p2 = _gcn_edges(src_r, dst_r, hw, dinv2d.reshape(NN))

    out = _stage_d(
        p2, h2s, gcn_bias.reshape(1, HID2), bn2_gamma.reshape(1, HID2),
        bn2_beta.reshape(1, HID2), gate_W, gate_b.reshape(1, 1),
        fc_W, fc_b.reshape(1, 1), batch.reshape(NN, 1))

    return out.reshape(NG)
